# Initial kernel scaffold; baseline (speedup 1.0000x reference)
import jax
import jax.numpy as jnp
from jax.experimental import pallas as pl
from jax.experimental.pallas import tpu as pltpu

_BI = 256


def _chamfer_block_kernel(gts_ref, predsT_ref, out_ref, colmin_ref, rowacc_ref):
    i = pl.program_id(1)
    n_i = pl.num_programs(1)

    @pl.when(i == 0)
    def _():
        rowacc_ref[0] = 0.0
        colmin_ref[...] = jnp.full_like(colmin_ref[...], jnp.inf)

    x = gts_ref[0]
    yT = predsT_ref[0]

    rx = jnp.sum(x * x, axis=1, keepdims=True)
    ry = jnp.sum(yT * yT, axis=0, keepdims=True)

    zz2 = jax.lax.dot_general(
        x * -2.0, yT, (((1,), (0,)), ((), ())),
        preferred_element_type=jnp.float32)

    p = zz2 + rx + ry

    rowacc_ref[0] += jnp.sum(jnp.min(p, axis=1))
    colmin_ref[...] = jnp.minimum(colmin_ref[...],
                                  jnp.min(p, axis=0, keepdims=True))

    @pl.when(i == n_i - 1)
    def _():
        total = rowacc_ref[0] + jnp.sum(colmin_ref[...])
        out_ref[...] = jnp.full((1, 1), total, dtype=jnp.float32)


def _chamfer(preds, gts, interpret=False):
    B, N, D = preds.shape
    predsT = jnp.transpose(preds, (0, 2, 1))
    out = pl.pallas_call(
        _chamfer_block_kernel,
        out_shape=jax.ShapeDtypeStruct((B, 1), jnp.float32),
        grid=(B, N // _BI),
        in_specs=[
            pl.BlockSpec((1, _BI, D), lambda b, i: (b, i, 0)),
            pl.BlockSpec((1, D, N), lambda b, i: (b, 0, 0)),
        ],
        out_specs=pl.BlockSpec((1, 1), lambda b, i: (b, 0)),
        scratch_shapes=[
            pltpu.VMEM((1, N), jnp.float32),
            pltpu.SMEM((1,), jnp.float32),
        ],
        compiler_params=pltpu.CompilerParams(
            dimension_semantics=("parallel", "arbitrary"),
        ),
        name="chamfer_loss",
        interpret=interpret,
    )(gts, predsT)
    return jnp.sum(out)


def kernel(preds, gts):
    return _chamfer(preds, gts)


# speedup vs baseline: 1.2126x; 1.2126x over previous
import jax
import jax.numpy as jnp
from jax.experimental import pallas as pl
from jax.experimental.pallas import tpu as pltpu

_BI = 256


def _chamfer_block_kernel(gts_ref, predsT_ref, out_ref, colmin_ref, rowacc_ref):
    i = pl.program_id(1)
    n_i = pl.num_programs(1)

    @pl.when(i == 0)
    def _():
        rowacc_ref[0] = 0.0
        colmin_ref[...] = jnp.full_like(colmin_ref[...], jnp.inf)

    x = gts_ref[0]
    yT = predsT_ref[0]

    rx = jnp.sum(x * x, axis=1, keepdims=True)
    ry = jnp.sum(yT * yT, axis=0, keepdims=True)

    zz2 = jax.lax.dot_general(
        x * -2.0, yT, (((1,), (0,)), ((), ())),
        preferred_element_type=jnp.float32)

    p = zz2 + rx + ry

    rowacc_ref[0] += jnp.sum(jnp.min(p, axis=1))
    colmin_ref[...] = jnp.minimum(colmin_ref[...],
                                  jnp.min(p, axis=0, keepdims=True))

    @pl.when(i == n_i - 1)
    def _():
        total = rowacc_ref[0] + jnp.sum(colmin_ref[...])
        out_ref[...] = jnp.full((1, 1, 1), total, dtype=jnp.float32)


def _chamfer(preds, gts, interpret=False):
    B, N, D = preds.shape
    predsT = jnp.transpose(preds, (0, 2, 1))
    out = pl.pallas_call(
        _chamfer_block_kernel,
        out_shape=jax.ShapeDtypeStruct((B, 1, 1), jnp.float32),
        grid=(B, N // _BI),
        in_specs=[
            pl.BlockSpec((1, _BI, D), lambda b, i: (b, i, 0)),
            pl.BlockSpec((1, D, N), lambda b, i: (b, 0, 0)),
        ],
        out_specs=pl.BlockSpec((1, 1, 1), lambda b, i: (b, 0, 0)),
        scratch_shapes=[
            pltpu.VMEM((1, N), jnp.float32),
            pltpu.SMEM((1,), jnp.float32),
        ],
        compiler_params=pltpu.CompilerParams(
            dimension_semantics=("parallel", "arbitrary"),
        ),
        name="chamfer_loss",
        interpret=interpret,
    )(gts, predsT)
    return jnp.sum(out)


def kernel(preds, gts):
    return _chamfer(preds, gts)
